# Initial kernel scaffold; baseline (speedup 1.0000x reference)
import jax
import jax.numpy as jnp
from jax.experimental import pallas as pl


def kernel(x, coef, spline_scale, base_scale):
    raise NotImplementedError("write your pallas kernel here")


# speedup vs baseline: 618.0001x; 618.0001x over previous
import jax
import jax.numpy as jnp
from jax.experimental import pallas as pl

N_INT = 16
_A = (
    (-1.0 / 6.0, 3.0 / 6.0, -3.0 / 6.0, 1.0 / 6.0),
    (3.0 / 6.0, -6.0 / 6.0, 3.0 / 6.0, 0.0),
    (-3.0 / 6.0, 0.0, 3.0 / 6.0, 0.0),
    (1.0 / 6.0, 4.0 / 6.0, 1.0 / 6.0, 0.0),
)


def _kan_kernel(x_ref, coef_ref, ss_ref, bs_ref, out_ref):
    x = x_ref[:]
    inv_b = 1.0 / x.shape[0]
    mean = jnp.sum(x, axis=0, keepdims=True) * inv_b
    xc = x - mean
    var = jnp.sum(xc * xc, axis=0, keepdims=True) * inv_b
    xn = xc * (jax.lax.rsqrt(var + 1e-5) * (1.0 / 3.0))

    base_in = xn * jax.nn.sigmoid(xn)
    acc = jnp.dot(base_in, bs_ref[:], preferred_element_type=jnp.float32)

    xcl = jnp.clip(xn, -1.0, 1.0)
    u = (xcl + 1.0) * (N_INT / 2.0)
    fi = jnp.floor(u)
    ci = jnp.minimum(fi, float(N_INT - 1))
    t = u - fi
    vs = []
    for j in range(4):
        a0, a1, a2, a3 = _A[0][j], _A[1][j], _A[2][j], _A[3][j]
        vs.append(((a0 * t + a1) * t + a2) * t + a3)

    ss = ss_ref[:]
    for k in range(N_INT + 3):
        pk = None
        for j in range(4):
            kj = k - j
            if 0 <= kj <= N_INT - 1:
                term = jnp.where(ci == float(kj), vs[j], 0.0)
                pk = term if pk is None else pk + term
        ck = coef_ref[:, k, :] * ss
        acc += jnp.dot(pk, ck, preferred_element_type=jnp.float32)

    out_ref[:] = acc


def kernel(x, coef, spline_scale, base_scale):
    return pl.pallas_call(
        _kan_kernel,
        out_shape=jax.ShapeDtypeStruct(x.shape, x.dtype),
    )(x, coef, spline_scale, base_scale)
